# baseline (speedup 1.0000x reference)
import jax
import jax.numpy as jnp
from jax import lax
from jax.experimental import pallas as pl
from jax.experimental.pallas import tpu as pltpu
from jax.experimental.pallas import tpu_sc as plsc

N_NODES = 10000
IN_DIM = 128
HID_DIM = 256
OUT_DIM = 64
N_EDGES = 320000

HALF = 128
NS = 16
NB = 168
BB = 128
SEG = 24
NSEG = NB // SEG
E_ACT = N_EDGES + N_NODES
E_PAD = NS * NB * BB

_NEG = -3e38


_GRID = 10
_BR = N_NODES // _GRID


def _att_outs(h, s_ref, d_ref, as_ref, ad_ref, ms_ref, md_ref):
    a_s = jnp.sum(h * s_ref[...], axis=1, keepdims=True)
    a_d = jnp.sum(h * d_ref[...], axis=1, keepdims=True)
    as_ref[...] = a_s
    ad_ref[...] = a_d
    ms_ref[...] = jnp.full((8, 1), jnp.max(a_s), jnp.float32)
    md_ref[...] = jnp.full((8, 1), jnp.max(a_d), jnp.float32)


def _tc_layer1_body(x_ref, w_ref, s_ref, d_ref,
                    h0_ref, h1_ref, as_ref, ad_ref, ms_ref, md_ref):
    h = jnp.dot(x_ref[...], w_ref[...], preferred_element_type=jnp.float32)
    h0_ref[...] = h[:, :HALF]
    h1_ref[...] = h[:, HALF:]
    _att_outs(h, s_ref, d_ref, as_ref, ad_ref, ms_ref, md_ref)


def _tc_layer2_body(o0_ref, o1_ref, b_ref, w_ref, s_ref, d_ref,
                    h0_ref, h1_ref, as_ref, ad_ref, ms_ref, md_ref):
    b = b_ref[...]
    x0 = jnp.maximum(o0_ref[...] + b[:, :HALF], 0.0)
    x1 = jnp.maximum(o1_ref[...] + b[:, HALF:], 0.0)
    w = w_ref[...]
    h = (jnp.dot(x0, w[:HALF, :], preferred_element_type=jnp.float32)
         + jnp.dot(x1, w[HALF:, :], preferred_element_type=jnp.float32))
    h0_ref[...] = h[:, :HALF]
    h1_ref[...] = h[:, HALF:]
    _att_outs(h, s_ref, d_ref, as_ref, ad_ref, ms_ref, md_ref)


def _tc_final_body(o0_ref, o1_ref, b_ref, w_ref, bf_ref, out_ref):
    b = b_ref[...]
    x0 = jnp.maximum(o0_ref[...] + b[:, :HALF], 0.0)
    x1 = jnp.maximum(o1_ref[...] + b[:, HALF:], 0.0)
    w = w_ref[...]
    o = (jnp.dot(x0, w[:HALF, :], preferred_element_type=jnp.float32)
         + jnp.dot(x1, w[HALF:, :], preferred_element_type=jnp.float32)
         + bf_ref[...])
    m = jnp.max(o, axis=1, keepdims=True)
    z = o - m
    out_ref[...] = z - jnp.log(jnp.sum(jnp.exp(z), axis=1, keepdims=True))


_LAYER_OUT_SHAPE = [
    jax.ShapeDtypeStruct((N_NODES, HALF), jnp.float32),
    jax.ShapeDtypeStruct((N_NODES, HALF), jnp.float32),
    jax.ShapeDtypeStruct((N_NODES, 1), jnp.float32),
    jax.ShapeDtypeStruct((N_NODES, 1), jnp.float32),
    jax.ShapeDtypeStruct((8 * _GRID, 1), jnp.float32),
    jax.ShapeDtypeStruct((8 * _GRID, 1), jnp.float32),
]

_LAYER_OUT_SPECS = [
    pl.BlockSpec((_BR, HALF), lambda i: (i, 0)),
    pl.BlockSpec((_BR, HALF), lambda i: (i, 0)),
    pl.BlockSpec((_BR, 1), lambda i: (i, 0)),
    pl.BlockSpec((_BR, 1), lambda i: (i, 0)),
    pl.BlockSpec((8, 1), lambda i: (i, 0)),
    pl.BlockSpec((8, 1), lambda i: (i, 0)),
]


def _tc_layer1(x, W, att_s, att_d):
    return pl.pallas_call(
        _tc_layer1_body,
        grid=(_GRID,),
        in_specs=[
            pl.BlockSpec((_BR, IN_DIM), lambda i: (i, 0)),
            pl.BlockSpec((IN_DIM, HID_DIM), lambda i: (0, 0)),
            pl.BlockSpec((1, HID_DIM), lambda i: (0, 0)),
            pl.BlockSpec((1, HID_DIM), lambda i: (0, 0)),
        ],
        out_specs=_LAYER_OUT_SPECS,
        out_shape=_LAYER_OUT_SHAPE,
    )(x, W, att_s[None, :], att_d[None, :])


def _tc_layer2(o0, o1, bias, W, att_s, att_d):
    return pl.pallas_call(
        _tc_layer2_body,
        grid=(_GRID,),
        in_specs=[
            pl.BlockSpec((_BR, HALF), lambda i: (i, 0)),
            pl.BlockSpec((_BR, HALF), lambda i: (i, 0)),
            pl.BlockSpec((1, HID_DIM), lambda i: (0, 0)),
            pl.BlockSpec((HID_DIM, HID_DIM), lambda i: (0, 0)),
            pl.BlockSpec((1, HID_DIM), lambda i: (0, 0)),
            pl.BlockSpec((1, HID_DIM), lambda i: (0, 0)),
        ],
        out_specs=_LAYER_OUT_SPECS,
        out_shape=_LAYER_OUT_SHAPE,
    )(o0, o1, bias[None, :], W, att_s[None, :], att_d[None, :])


def _tc_final(o0, o1, bias, Wfc, bfc):
    return pl.pallas_call(
        _tc_final_body,
        grid=(_GRID,),
        in_specs=[
            pl.BlockSpec((_BR, HALF), lambda i: (i, 0)),
            pl.BlockSpec((_BR, HALF), lambda i: (i, 0)),
            pl.BlockSpec((1, HID_DIM), lambda i: (0, 0)),
            pl.BlockSpec((HID_DIM, OUT_DIM), lambda i: (0, 0)),
            pl.BlockSpec((1, OUT_DIM), lambda i: (0, 0)),
        ],
        out_specs=pl.BlockSpec((_BR, OUT_DIM), lambda i: (i, 0)),
        out_shape=jax.ShapeDtypeStruct((N_NODES, OUT_DIM), jnp.float32),
    )(o0, o1, bias[None, :], Wfc, bfc[None, :])


def _sc_gat_body(h0, h1, asc, adc, mxs, mxd, src3, dst3, out,
                 srcv, dstv, ev, db, gb0, gb1, mbuf,
                 sem0, sem1, semc0, semc1, sem_a, sem_d,
                 asrc_sh, adst_sh, denom_sh, out_sh):
    c = lax.axis_index("c")
    s = lax.axis_index("s")

    @pl.when(s == 0)
    def _():
        pltpu.sync_copy(asc, asrc_sh)

    @pl.when(s == 1)
    def _():
        pltpu.sync_copy(adc, adst_sh)

    pltpu.sync_copy(mxs, mbuf.at[pl.ds(0, 80)])
    pltpu.sync_copy(mxd, mbuf.at[pl.ds(128, 80)])
    neg16 = jnp.full((16,), _NEG, jnp.float32)
    acc1 = neg16
    acc2 = neg16
    for j in range(5):
        acc1 = jnp.maximum(acc1, mbuf[pl.ds(j * 16, 16)])
        acc2 = jnp.maximum(acc2, mbuf[pl.ds(128 + j * 16, 16)])

    iota16 = lax.iota(jnp.int32, 16)

    def _allmax(v):
        for sh in (8, 4, 2, 1):
            mbuf[pl.ds(0, 16)] = v
            v = jnp.maximum(v, plsc.load_gather(mbuf, [iota16 ^ sh]))
        return v

    mtot = _allmax(acc1) + _allmax(acc2)
    m16 = jnp.maximum(mtot, 0.2 * mtot)

    zero16 = jnp.zeros((16,), jnp.float32)
    for k in range(8):
        db[0, pl.ds(k * 16, 16)] = zero16

    def _zg(i, carry):
        for k in range(8):
            gb0[i, pl.ds(k * 16, 16)] = zero16
        return carry

    lax.fori_loop(0, 8, _zg, 0)

    @pl.when(s < 10)
    def _():
        def _zd(k, carry):
            pltpu.sync_copy(db.at[0],
                            denom_sh.at[pl.ds(s * 1024 + k * 128, 128)])
            return carry

        lax.fori_loop(0, 8, _zd, 0)

    def _zout(r, carry):
        pltpu.sync_copy(gb0.at[pl.ds(0, 8)],
                        out_sh.at[pl.ds(s * 640 + r * 8, 8)])
        return carry

    lax.fori_loop(0, jnp.where(s < 15, 80, 50), _zout, 0)

    plsc.subcore_barrier()

    def _gstart(idx_ref, buf, gsem):
        @pl.when(c == 0)
        def _():
            pltpu.async_copy(h0.at[idx_ref], buf, gsem)

        @pl.when(c == 1)
        def _():
            pltpu.async_copy(h1.at[idx_ref], buf, gsem)

    def _gwait(buf, gsem):
        pltpu.make_async_copy(h0.at[srcv.at[0]], buf, gsem).wait()

    def _seg(seg, carry):
        pltpu.sync_copy(src3.at[s].at[pl.ds(seg * SEG, SEG)], srcv)
        pltpu.sync_copy(dst3.at[s].at[pl.ds(seg * SEG, SEG)], dstv)

        def _fire_a(r, carry2):
            pltpu.async_copy(asrc_sh.at[srcv.at[r]], ev.at[r], sem_a)
            pltpu.async_copy(adst_sh.at[dstv.at[r]], db.at[r], sem_a)
            return carry2

        lax.fori_loop(0, SEG, _fire_a, 0)

        def _drain_a(r, carry2):
            pltpu.make_async_copy(asrc_sh.at[srcv.at[0]], ev.at[0],
                                  sem_a).wait()
            pltpu.make_async_copy(asrc_sh.at[srcv.at[0]], db.at[0],
                                  sem_a).wait()
            return carry2

        lax.fori_loop(0, SEG, _drain_a, 0)

        def _zrow(r, carry2):
            for k in range(BB // 16):
                e = ev[r, pl.ds(k * 16, 16)] + db[r, pl.ds(k * 16, 16)]
                e = jnp.maximum(e, 0.2 * e)
                z = jnp.exp(e - m16)
                eid = (s * NB + seg * SEG + r) * BB + k * 16 + iota16
                z = jnp.where(eid < E_ACT, z, 0.0)
                ev[r, pl.ds(k * 16, 16)] = z
            pltpu.async_copy(ev.at[r], denom_sh.at[dstv.at[r]], sem_d,
                             add=True)
            return carry2

        lax.fori_loop(0, SEG, _zrow, 0)

        def _scale(r, buf):
            @plsc.parallel_loop(0, BB // 16, unroll=4)
            def _scale_g(g):
                zlf = ev[r, pl.ds(g * 16, 16)]
                for e16 in range(16):
                    zj = jnp.take_along_axis(
                        zlf, jnp.full((16,), e16, jnp.int32), axis=0)
                    e = g * 16 + e16
                    for kk in range(HALF // 16):
                        buf[e, pl.ds(kk * 16, 16)] = (
                            buf[e, pl.ds(kk * 16, 16)] * zj)

        def _cstart(r, buf, csem):
            pltpu.async_copy(buf, out_sh.at[dstv.at[r]], csem, add=True)

        def _cwait(buf, csem):
            pltpu.make_async_copy(h0.at[srcv.at[0]], buf, csem).wait()

        _gstart(srcv.at[0], gb0, sem0)
        _gstart(srcv.at[1], gb1, sem1)

        def _pair(p, carry2):
            r0 = 2 * p
            _gwait(gb0, sem0)
            _scale(r0, gb0)
            _cstart(r0, gb0, semc0)

            @pl.when(r0 + 2 < SEG)
            def _():
                _cwait(gb0, semc0)
                _gstart(srcv.at[r0 + 2], gb0, sem0)

            _gwait(gb1, sem1)
            _scale(r0 + 1, gb1)
            _cstart(r0 + 1, gb1, semc1)

            @pl.when(r0 + 3 < SEG)
            def _():
                _cwait(gb1, semc1)
                _gstart(srcv.at[r0 + 3], gb1, sem1)

            return carry2

        lax.fori_loop(0, SEG // 2, _pair, 0)

        _cwait(gb0, semc0)
        _cwait(gb1, semc1)

        def _drain_d(r, carry2):
            pltpu.make_async_copy(asrc_sh.at[srcv.at[0]], ev.at[0],
                                  sem_d).wait()
            return carry2

        lax.fori_loop(0, SEG, _drain_d, 0)
        return carry

    lax.fori_loop(0, NSEG, _seg, 0)

    plsc.subcore_barrier()

    def _normwrite(b, carry):
        row0 = s * 640 + b * 80
        pltpu.sync_copy(denom_sh.at[pl.ds(row0, 80)],
                        db.at[0].at[pl.ds(0, 80)])
        pltpu.sync_copy(out_sh.at[pl.ds(row0, 80)], gb0.at[pl.ds(0, 80)])

        @plsc.parallel_loop(0, 5)
        def _norm(g):
            dv = db[0, pl.ds(g * 16, 16)]
            rv = 1.0 / dv
            for e16 in range(16):
                rj = jnp.take_along_axis(
                    rv, jnp.full((16,), e16, jnp.int32), axis=0)
                rr = g * 16 + e16
                for kk in range(HALF // 16):
                    gb0[rr, pl.ds(kk * 16, 16)] = (
                        gb0[rr, pl.ds(kk * 16, 16)] * rj)

        pltpu.sync_copy(gb0.at[pl.ds(0, 80)], out.at[c].at[pl.ds(row0, 80)])
        return carry

    lax.fori_loop(0, jnp.where(s < 15, 8, 5), _normwrite, 0)


def _sc_gat(h0, h1, a_src, a_dst, mxs, mxd, src3, dst3):
    mesh = plsc.VectorSubcoreMesh(core_axis_name="c", subcore_axis_name="s")
    f = pl.kernel(
        _sc_gat_body,
        out_type=jax.ShapeDtypeStruct((2, N_NODES, HALF), jnp.float32),
        mesh=mesh,
        compiler_params=pltpu.CompilerParams(needs_layout_passes=False),
        scratch_types=[
            pltpu.VMEM((SEG, BB), jnp.int32),
            pltpu.VMEM((SEG, BB), jnp.int32),
            pltpu.VMEM((SEG, BB), jnp.float32),
            pltpu.VMEM((SEG, BB), jnp.float32),
            pltpu.VMEM((BB, HALF), jnp.float32),
            pltpu.VMEM((BB, HALF), jnp.float32),
            pltpu.VMEM((256,), jnp.float32),
            pltpu.SemaphoreType.DMA,
            pltpu.SemaphoreType.DMA,
            pltpu.SemaphoreType.DMA,
            pltpu.SemaphoreType.DMA,
            pltpu.SemaphoreType.DMA,
            pltpu.SemaphoreType.DMA,
            pltpu.VMEM_SHARED((N_NODES,), jnp.float32),
            pltpu.VMEM_SHARED((N_NODES,), jnp.float32),
            pltpu.VMEM_SHARED((10240,), jnp.float32),
            pltpu.VMEM_SHARED((N_NODES, HALF), jnp.float32),
        ],
    )
    return f(h0, h1, a_src, a_dst, mxs, mxd, src3, dst3)


def kernel(x, edge_index, W1, att_src1, att_dst1, b1,
           W2, att_src2, att_dst2, b2, Wfc, bfc):
    ei = edge_index.astype(jnp.int32)
    loop = jnp.arange(N_NODES, dtype=jnp.int32)
    pad = jnp.arange(E_PAD - E_ACT, dtype=jnp.int32) % N_NODES
    src3 = jnp.concatenate([ei[0], loop, pad]).reshape(NS, NB, BB)
    dst3 = jnp.concatenate([ei[1], loop, pad]).reshape(NS, NB, BB)

    h0, h1, as1, ad1, ms1, md1 = _tc_layer1(x, W1, att_src1, att_dst1)
    o1 = _sc_gat(h0, h1, as1.reshape(N_NODES), ad1.reshape(N_NODES),
                 ms1.reshape(80), md1.reshape(80), src3, dst3)
    h0, h1, as2, ad2, ms2, md2 = _tc_layer2(o1[0], o1[1], b1, W2,
                                            att_src2, att_dst2)
    o2 = _sc_gat(h0, h1, as2.reshape(N_NODES), ad2.reshape(N_NODES),
                 ms2.reshape(80), md2.reshape(80), src3, dst3)
    return _tc_final(o2[0], o2[1], b2, Wfc, bfc)


# speedup vs baseline: 1.0117x; 1.0117x over previous
import jax
import jax.numpy as jnp
from jax import lax
from jax.experimental import pallas as pl
from jax.experimental.pallas import tpu as pltpu
from jax.experimental.pallas import tpu_sc as plsc

N_NODES = 10000
IN_DIM = 128
HID_DIM = 256
OUT_DIM = 64
N_EDGES = 320000

HALF = 128
NS = 16
NB = 168
BB = 128
SEG = 24
NSEG = NB // SEG
E_ACT = N_EDGES + N_NODES
E_PAD = NS * NB * BB

_NEG = -3e38


_GRID = 10
_BR = N_NODES // _GRID


def _att_outs(h, s_ref, d_ref, as_ref, ad_ref, ms_ref, md_ref):
    a_s = jnp.sum(h * s_ref[...], axis=1, keepdims=True)
    a_d = jnp.sum(h * d_ref[...], axis=1, keepdims=True)
    as_ref[...] = a_s
    ad_ref[...] = a_d
    ms_ref[...] = jnp.full((8, 1), jnp.max(a_s), jnp.float32)
    md_ref[...] = jnp.full((8, 1), jnp.max(a_d), jnp.float32)


def _tc_layer1_body(x_ref, w_ref, s_ref, d_ref,
                    h0_ref, h1_ref, as_ref, ad_ref, ms_ref, md_ref):
    h = jnp.dot(x_ref[...], w_ref[...], preferred_element_type=jnp.float32)
    h0_ref[...] = h[:, :HALF]
    h1_ref[...] = h[:, HALF:]
    _att_outs(h, s_ref, d_ref, as_ref, ad_ref, ms_ref, md_ref)


def _tc_layer2_body(o0_ref, o1_ref, b_ref, w_ref, s_ref, d_ref,
                    h0_ref, h1_ref, as_ref, ad_ref, ms_ref, md_ref):
    b = b_ref[...]
    x0 = jnp.maximum(o0_ref[...] + b[:, :HALF], 0.0)
    x1 = jnp.maximum(o1_ref[...] + b[:, HALF:], 0.0)
    w = w_ref[...]
    h = (jnp.dot(x0, w[:HALF, :], preferred_element_type=jnp.float32)
         + jnp.dot(x1, w[HALF:, :], preferred_element_type=jnp.float32))
    h0_ref[...] = h[:, :HALF]
    h1_ref[...] = h[:, HALF:]
    _att_outs(h, s_ref, d_ref, as_ref, ad_ref, ms_ref, md_ref)


def _tc_final_body(o0_ref, o1_ref, b_ref, w_ref, bf_ref, out_ref):
    b = b_ref[...]
    x0 = jnp.maximum(o0_ref[...] + b[:, :HALF], 0.0)
    x1 = jnp.maximum(o1_ref[...] + b[:, HALF:], 0.0)
    w = w_ref[...]
    o = (jnp.dot(x0, w[:HALF, :], preferred_element_type=jnp.float32)
         + jnp.dot(x1, w[HALF:, :], preferred_element_type=jnp.float32)
         + bf_ref[...])
    m = jnp.max(o, axis=1, keepdims=True)
    z = o - m
    out_ref[...] = z - jnp.log(jnp.sum(jnp.exp(z), axis=1, keepdims=True))


_LAYER_OUT_SHAPE = [
    jax.ShapeDtypeStruct((N_NODES, HALF), jnp.float32),
    jax.ShapeDtypeStruct((N_NODES, HALF), jnp.float32),
    jax.ShapeDtypeStruct((N_NODES, 1), jnp.float32),
    jax.ShapeDtypeStruct((N_NODES, 1), jnp.float32),
    jax.ShapeDtypeStruct((8 * _GRID, 1), jnp.float32),
    jax.ShapeDtypeStruct((8 * _GRID, 1), jnp.float32),
]

_LAYER_OUT_SPECS = [
    pl.BlockSpec((_BR, HALF), lambda i: (i, 0)),
    pl.BlockSpec((_BR, HALF), lambda i: (i, 0)),
    pl.BlockSpec((_BR, 1), lambda i: (i, 0)),
    pl.BlockSpec((_BR, 1), lambda i: (i, 0)),
    pl.BlockSpec((8, 1), lambda i: (i, 0)),
    pl.BlockSpec((8, 1), lambda i: (i, 0)),
]


def _tc_layer1(x, W, att_s, att_d):
    return pl.pallas_call(
        _tc_layer1_body,
        grid=(_GRID,),
        in_specs=[
            pl.BlockSpec((_BR, IN_DIM), lambda i: (i, 0)),
            pl.BlockSpec((IN_DIM, HID_DIM), lambda i: (0, 0)),
            pl.BlockSpec((1, HID_DIM), lambda i: (0, 0)),
            pl.BlockSpec((1, HID_DIM), lambda i: (0, 0)),
        ],
        out_specs=_LAYER_OUT_SPECS,
        out_shape=_LAYER_OUT_SHAPE,
    )(x, W, att_s[None, :], att_d[None, :])


def _tc_layer2(o0, o1, bias, W, att_s, att_d):
    return pl.pallas_call(
        _tc_layer2_body,
        grid=(_GRID,),
        in_specs=[
            pl.BlockSpec((_BR, HALF), lambda i: (i, 0)),
            pl.BlockSpec((_BR, HALF), lambda i: (i, 0)),
            pl.BlockSpec((1, HID_DIM), lambda i: (0, 0)),
            pl.BlockSpec((HID_DIM, HID_DIM), lambda i: (0, 0)),
            pl.BlockSpec((1, HID_DIM), lambda i: (0, 0)),
            pl.BlockSpec((1, HID_DIM), lambda i: (0, 0)),
        ],
        out_specs=_LAYER_OUT_SPECS,
        out_shape=_LAYER_OUT_SHAPE,
    )(o0, o1, bias[None, :], W, att_s[None, :], att_d[None, :])


def _tc_final(o0, o1, bias, Wfc, bfc):
    return pl.pallas_call(
        _tc_final_body,
        grid=(_GRID,),
        in_specs=[
            pl.BlockSpec((_BR, HALF), lambda i: (i, 0)),
            pl.BlockSpec((_BR, HALF), lambda i: (i, 0)),
            pl.BlockSpec((1, HID_DIM), lambda i: (0, 0)),
            pl.BlockSpec((HID_DIM, OUT_DIM), lambda i: (0, 0)),
            pl.BlockSpec((1, OUT_DIM), lambda i: (0, 0)),
        ],
        out_specs=pl.BlockSpec((_BR, OUT_DIM), lambda i: (i, 0)),
        out_shape=jax.ShapeDtypeStruct((N_NODES, OUT_DIM), jnp.float32),
    )(o0, o1, bias[None, :], Wfc, bfc[None, :])


def _sc_gat_body(h0, h1, asc, adc, mxs, mxd, src3, dst3, out,
                 srcv, dstv, ev, db, gb0, gb1, mbuf, sem0, sem1,
                 asrc_sh, adst_sh, denom_sh, out_sh):
    c = lax.axis_index("c")
    s = lax.axis_index("s")

    @pl.when(s == 0)
    def _():
        pltpu.sync_copy(asc, asrc_sh)

    @pl.when(s == 1)
    def _():
        pltpu.sync_copy(adc, adst_sh)

    pltpu.sync_copy(mxs, mbuf.at[pl.ds(0, 80)])
    pltpu.sync_copy(mxd, mbuf.at[pl.ds(128, 80)])
    neg16 = jnp.full((16,), _NEG, jnp.float32)
    acc1 = neg16
    acc2 = neg16
    for j in range(5):
        acc1 = jnp.maximum(acc1, mbuf[pl.ds(j * 16, 16)])
        acc2 = jnp.maximum(acc2, mbuf[pl.ds(128 + j * 16, 16)])

    iota16 = lax.iota(jnp.int32, 16)

    def _allmax(v):
        for sh in (8, 4, 2, 1):
            mbuf[pl.ds(0, 16)] = v
            v = jnp.maximum(v, plsc.load_gather(mbuf, [iota16 ^ sh]))
        return v

    mtot = _allmax(acc1) + _allmax(acc2)
    m16 = jnp.maximum(mtot, 0.2 * mtot)

    zero16 = jnp.zeros((16,), jnp.float32)
    for k in range(8):
        db[0, pl.ds(k * 16, 16)] = zero16

    def _zg(i, carry):
        for k in range(8):
            gb0[i, pl.ds(k * 16, 16)] = zero16
        return carry

    lax.fori_loop(0, 8, _zg, 0)

    @pl.when(s < 10)
    def _():
        def _zd(k, carry):
            pltpu.sync_copy(db.at[0],
                            denom_sh.at[pl.ds(s * 1024 + k * 128, 128)])
            return carry

        lax.fori_loop(0, 8, _zd, 0)

    def _zout(r, carry):
        pltpu.sync_copy(gb0.at[pl.ds(0, 8)],
                        out_sh.at[pl.ds(s * 640 + r * 8, 8)])
        return carry

    lax.fori_loop(0, jnp.where(s < 15, 80, 50), _zout, 0)

    plsc.subcore_barrier()

    def _gstart(idx_ref, buf, gsem):
        @pl.when(c == 0)
        def _():
            pltpu.async_copy(h0.at[idx_ref], buf, gsem)

        @pl.when(c == 1)
        def _():
            pltpu.async_copy(h1.at[idx_ref], buf, gsem)

    def _gwait(buf, gsem):
        pltpu.make_async_copy(h0.at[srcv.at[0]], buf, gsem).wait()

    def _seg(seg, carry):
        pltpu.sync_copy(src3.at[s].at[pl.ds(seg * SEG, SEG)], srcv)
        pltpu.sync_copy(dst3.at[s].at[pl.ds(seg * SEG, SEG)], dstv)

        def _row_front(r):
            pltpu.sync_copy(asrc_sh.at[srcv.at[r]], ev.at[r])
            pltpu.sync_copy(adst_sh.at[dstv.at[r]], ev.at[r], add=True)
            for k in range(BB // 16):
                e = ev[r, pl.ds(k * 16, 16)]
                e = jnp.maximum(e, 0.2 * e)
                z = jnp.exp(e - m16)
                eid = (s * NB + seg * SEG + r) * BB + k * 16 + iota16
                z = jnp.where(eid < E_ACT, z, 0.0)
                ev[r, pl.ds(k * 16, 16)] = z
            pltpu.sync_copy(ev.at[r], denom_sh.at[dstv.at[r]], add=True)

        def _scale_scatter(r, buf):
            @plsc.parallel_loop(0, BB // 16, unroll=2)
            def _scale(g):
                zlf = ev[r, pl.ds(g * 16, 16)]
                for e16 in range(16):
                    zj = jnp.take_along_axis(
                        zlf, jnp.full((16,), e16, jnp.int32), axis=0)
                    e = g * 16 + e16
                    for kk in range(HALF // 16):
                        buf[e, pl.ds(kk * 16, 16)] = (
                            buf[e, pl.ds(kk * 16, 16)] * zj)
            pltpu.sync_copy(buf, out_sh.at[dstv.at[r]], add=True)

        _gstart(srcv.at[0], gb0, sem0)

        def _pair(p, carry2):
            r0 = 2 * p
            _gstart(srcv.at[r0 + 1], gb1, sem1)
            _row_front(r0)
            _gwait(gb0, sem0)
            _scale_scatter(r0, gb0)

            @pl.when(r0 + 2 < SEG)
            def _():
                _gstart(srcv.at[r0 + 2], gb0, sem0)

            _row_front(r0 + 1)
            _gwait(gb1, sem1)
            _scale_scatter(r0 + 1, gb1)
            return carry2

        lax.fori_loop(0, SEG // 2, _pair, 0)
        return carry

    lax.fori_loop(0, NSEG, _seg, 0)

    plsc.subcore_barrier()

    def _normwrite(b, carry):
        row0 = s * 640 + b * 80
        pltpu.sync_copy(denom_sh.at[pl.ds(row0, 80)],
                        db.at[0].at[pl.ds(0, 80)])
        pltpu.sync_copy(out_sh.at[pl.ds(row0, 80)], gb0.at[pl.ds(0, 80)])

        @plsc.parallel_loop(0, 5)
        def _norm(g):
            dv = db[0, pl.ds(g * 16, 16)]
            rv = 1.0 / dv
            for e16 in range(16):
                rj = jnp.take_along_axis(
                    rv, jnp.full((16,), e16, jnp.int32), axis=0)
                rr = g * 16 + e16
                for kk in range(HALF // 16):
                    gb0[rr, pl.ds(kk * 16, 16)] = (
                        gb0[rr, pl.ds(kk * 16, 16)] * rj)

        pltpu.sync_copy(gb0.at[pl.ds(0, 80)], out.at[c].at[pl.ds(row0, 80)])
        return carry

    lax.fori_loop(0, jnp.where(s < 15, 8, 5), _normwrite, 0)


def _sc_gat(h0, h1, a_src, a_dst, mxs, mxd, src3, dst3):
    mesh = plsc.VectorSubcoreMesh(core_axis_name="c", subcore_axis_name="s")
    f = pl.kernel(
        _sc_gat_body,
        out_type=jax.ShapeDtypeStruct((2, N_NODES, HALF), jnp.float32),
        mesh=mesh,
        compiler_params=pltpu.CompilerParams(needs_layout_passes=False),
        scratch_types=[
            pltpu.VMEM((SEG, BB), jnp.int32),
            pltpu.VMEM((SEG, BB), jnp.int32),
            pltpu.VMEM((SEG, BB), jnp.float32),
            pltpu.VMEM((SEG, BB), jnp.float32),
            pltpu.VMEM((BB, HALF), jnp.float32),
            pltpu.VMEM((BB, HALF), jnp.float32),
            pltpu.VMEM((256,), jnp.float32),
            pltpu.SemaphoreType.DMA,
            pltpu.SemaphoreType.DMA,
            pltpu.VMEM_SHARED((N_NODES,), jnp.float32),
            pltpu.VMEM_SHARED((N_NODES,), jnp.float32),
            pltpu.VMEM_SHARED((10240,), jnp.float32),
            pltpu.VMEM_SHARED((N_NODES, HALF), jnp.float32),
        ],
    )
    return f(h0, h1, a_src, a_dst, mxs, mxd, src3, dst3)


def kernel(x, edge_index, W1, att_src1, att_dst1, b1,
           W2, att_src2, att_dst2, b2, Wfc, bfc):
    ei = edge_index.astype(jnp.int32)
    loop = jnp.arange(N_NODES, dtype=jnp.int32)
    pad = jnp.arange(E_PAD - E_ACT, dtype=jnp.int32) % N_NODES
    src3 = jnp.concatenate([ei[0], loop, pad]).reshape(NS, NB, BB)
    dst3 = jnp.concatenate([ei[1], loop, pad]).reshape(NS, NB, BB)

    h0, h1, as1, ad1, ms1, md1 = _tc_layer1(x, W1, att_src1, att_dst1)
    o1 = _sc_gat(h0, h1, as1.reshape(N_NODES), ad1.reshape(N_NODES),
                 ms1.reshape(80), md1.reshape(80), src3, dst3)
    h0, h1, as2, ad2, ms2, md2 = _tc_layer2(o1[0], o1[1], b1, W2,
                                            att_src2, att_dst2)
    o2 = _sc_gat(h0, h1, as2.reshape(N_NODES), ad2.reshape(N_NODES),
                 ms2.reshape(80), md2.reshape(80), src3, dst3)
    return _tc_final(o2[0], o2[1], b2, Wfc, bfc)
